# baseline (speedup 1.0000x reference)
import jax
import jax.numpy as jnp
from jax import lax
from jax.experimental import pallas as pl
from jax.experimental.pallas import tpu as pltpu
from jax.experimental.pallas import tpu_sc as plsc

_NC = 2
_NS = 16
_NW = _NC * _NS
_L = 16

_M, _N = 16384, 200
_ROWS_PER_W = _M // _NW
_R = 64
_NCHUNK = _ROWS_PER_W // _R
_COLS = tuple(range(0, _N - _L + 1, _L)) + (_N - _L,)


_GATHER_DNUMS = lax.GatherDimensionNumbers(
    offset_dims=(), collapsed_slice_dims=(0,), start_index_map=(0,))


def _lookup(tbl, idx):
    return lax.gather(tbl, idx[:, None], dimension_numbers=_GATHER_DNUMS,
                      slice_sizes=(1,),
                      mode=lax.GatherScatterMode.PROMISE_IN_BOUNDS)


def _body(tbl_hbm, idx_hbm, out_hbm, tbl_v, idx_v, out_v):
    wid = lax.axis_index("s") * _NC + lax.axis_index("c")
    pltpu.sync_copy(tbl_hbm, tbl_v)
    tbl = tbl_v[...]
    row0 = wid * _ROWS_PER_W

    def chunk(ch, carry):
        base = row0 + ch * _R
        pltpu.sync_copy(idx_hbm.at[pl.ds(base, _R)], idx_v)

        @plsc.parallel_loop(0, _R, step=1, unroll=2)
        def row(r):
            for col in _COLS:
                idx = idx_v[r, pl.ds(col, _L)]
                out_v[r, pl.ds(col, _L)] = _lookup(tbl, idx)
        pltpu.sync_copy(out_v, out_hbm.at[pl.ds(base, _R)])
        return carry

    lax.fori_loop(0, _NCHUNK, chunk, 0)


def kernel(inputs, values):
    table16 = jnp.concatenate(
        [values.astype(jnp.float32),
         jnp.full((_L - values.shape[0],), 63.0, jnp.float32)])
    fn = pl.kernel(
        _body,
        out_type=jax.ShapeDtypeStruct((_M, _N), jnp.float32),
        mesh=plsc.VectorSubcoreMesh(
            core_axis_name="c", subcore_axis_name="s", num_cores=_NC),
        scratch_types=[
            pltpu.VMEM((_L,), jnp.float32),
            pltpu.VMEM((_R, _N), jnp.int32),
            pltpu.VMEM((_R, _N), jnp.float32),
        ],
        compiler_params=pltpu.CompilerParams(use_tc_tiling_on_sc=True),
    )
    return fn(table16, inputs)


# speedup vs baseline: 2.1340x; 2.1340x over previous
import jax
import jax.numpy as jnp
from jax import lax
from jax.experimental import pallas as pl
from jax.experimental.pallas import tpu as pltpu
from jax.experimental.pallas import tpu_sc as plsc

_NC = 2
_NS = 16
_NW = _NC * _NS
_L = 16

_M, _N = 16384, 200
_W = _M // _NW
_CH = 128
_NCH = _W // _CH

_GATHER_DNUMS = lax.GatherDimensionNumbers(
    offset_dims=(), collapsed_slice_dims=(0,), start_index_map=(0,))


def _lookup(tbl, idx):
    return lax.gather(tbl, idx[:, None], dimension_numbers=_GATHER_DNUMS,
                      slice_sizes=(1,),
                      mode=lax.GatherScatterMode.PROMISE_IN_BOUNDS)


def _body(tbl_hbm, x_hbm, out_hbm, tbl_v, idx0, idx1, o0, o1,
          si0, si1, so0, so1):
    wid = lax.axis_index("s") * _NC + lax.axis_index("c")
    pltpu.sync_copy(tbl_hbm, tbl_v)
    tbl = tbl_v[...]
    col0 = wid * _W
    idx_b, out_b = (idx0, idx1), (o0, o1)
    sin, sout = (si0, si1), (so0, so1)

    in_cp = [None] * _NCH
    out_cp = [None] * _NCH
    in_cp[0] = pltpu.async_copy(x_hbm.at[:, pl.ds(col0, _CH)], idx_b[0], sin[0])
    for ch in range(_NCH):
        b = ch % 2
        in_cp[ch].wait()
        if ch + 1 < _NCH:
            nb = (ch + 1) % 2
            in_cp[ch + 1] = pltpu.async_copy(
                x_hbm.at[:, pl.ds(col0 + (ch + 1) * _CH, _CH)],
                idx_b[nb], sin[nb])
        if ch >= 2:
            out_cp[ch - 2].wait()

        @plsc.parallel_loop(0, _N, step=1, unroll=2)
        def row(r):
            for g in range(_CH // _L):
                iv = idx_b[b][r, pl.ds(g * _L, _L)]
                out_b[b][r, pl.ds(g * _L, _L)] = _lookup(tbl, iv)

        out_cp[ch] = pltpu.async_copy(
            out_b[b], out_hbm.at[:, pl.ds(col0 + ch * _CH, _CH)], sout[b])
    out_cp[_NCH - 2].wait()
    out_cp[_NCH - 1].wait()


def kernel(inputs, values):
    table16 = jnp.concatenate(
        [values.astype(jnp.float32),
         jnp.full((_L - values.shape[0],), 63.0, jnp.float32)])
    fn = pl.kernel(
        _body,
        out_type=jax.ShapeDtypeStruct((_N, _M), jnp.float32),
        mesh=plsc.VectorSubcoreMesh(
            core_axis_name="c", subcore_axis_name="s", num_cores=_NC),
        scratch_types=[
            pltpu.VMEM((_L,), jnp.float32),
            pltpu.VMEM((_N, _CH), jnp.int32),
            pltpu.VMEM((_N, _CH), jnp.int32),
            pltpu.VMEM((_N, _CH), jnp.float32),
            pltpu.VMEM((_N, _CH), jnp.float32),
            pltpu.SemaphoreType.DMA,
            pltpu.SemaphoreType.DMA,
            pltpu.SemaphoreType.DMA,
            pltpu.SemaphoreType.DMA,
        ],
        compiler_params=pltpu.CompilerParams(use_tc_tiling_on_sc=True),
    )
    return fn(table16, inputs.T).T


# speedup vs baseline: 2.2133x; 1.0371x over previous
import jax
import jax.numpy as jnp
from jax import lax
from jax.experimental import pallas as pl
from jax.experimental.pallas import tpu as pltpu
from jax.experimental.pallas import tpu_sc as plsc

_NC = 2
_NS = 16
_NW = _NC * _NS
_L = 16

_M, _N = 16384, 200
_W = _M // _NW
_CH = 128
_NCH = _W // _CH
_GPR = _CH // _L
_NG = _N * _GPR

_GATHER_DNUMS = lax.GatherDimensionNumbers(
    offset_dims=(), collapsed_slice_dims=(0,), start_index_map=(0,))


def _lookup(tbl, idx):
    return lax.gather(tbl, idx[:, None], dimension_numbers=_GATHER_DNUMS,
                      slice_sizes=(1,),
                      mode=lax.GatherScatterMode.PROMISE_IN_BOUNDS)


def _body(val_hbm, x_hbm, out_hbm, tbl_v, idx0, idx1, o0, o1,
          si0, si1, so0, so1, st):
    wid = lax.axis_index("s") * _NC + lax.axis_index("c")
    col0 = wid * _W
    idx_b, out_b = (idx0, idx1), (o0, o1)
    sin, sout = (si0, si1), (so0, so1)

    tbl_cp = pltpu.async_copy(val_hbm, tbl_v.at[pl.ds(0, 3)], st)
    in_cp = [None] * _NCH
    out_cp = [None] * _NCH
    in_cp[0] = pltpu.async_copy(x_hbm.at[:, pl.ds(col0, _CH)], idx_b[0], sin[0])
    tbl_cp.wait()
    tbl = jnp.where(lax.iota(jnp.int32, _L) < 3, tbl_v[...], 63.0)
    for ch in range(_NCH):
        b = ch % 2
        in_cp[ch].wait()
        if ch + 1 < _NCH:
            nb = (ch + 1) % 2
            in_cp[ch + 1] = pltpu.async_copy(
                x_hbm.at[:, pl.ds(col0 + (ch + 1) * _CH, _CH)],
                idx_b[nb], sin[nb])
        if ch >= 2:
            out_cp[ch - 2].wait()

        @plsc.parallel_loop(0, _NG, step=1, unroll=4)
        def grp(g):
            r = lax.shift_right_logical(g, 3)
            c = lax.shift_left(lax.bitwise_and(g, _GPR - 1), 4)
            iv = idx_b[b][r, pl.ds(c, _L)]
            out_b[b][r, pl.ds(c, _L)] = _lookup(tbl, iv)

        out_cp[ch] = pltpu.async_copy(
            out_b[b], out_hbm.at[:, pl.ds(col0 + ch * _CH, _CH)], sout[b])
    out_cp[_NCH - 2].wait()
    out_cp[_NCH - 1].wait()


def kernel(inputs, values):
    fn = pl.kernel(
        _body,
        out_type=jax.ShapeDtypeStruct((_N, _M), jnp.float32),
        mesh=plsc.VectorSubcoreMesh(
            core_axis_name="c", subcore_axis_name="s", num_cores=_NC),
        scratch_types=[
            pltpu.VMEM((_L,), jnp.float32),
            pltpu.VMEM((_N, _CH), jnp.int32),
            pltpu.VMEM((_N, _CH), jnp.int32),
            pltpu.VMEM((_N, _CH), jnp.float32),
            pltpu.VMEM((_N, _CH), jnp.float32),
            pltpu.SemaphoreType.DMA,
            pltpu.SemaphoreType.DMA,
            pltpu.SemaphoreType.DMA,
            pltpu.SemaphoreType.DMA,
            pltpu.SemaphoreType.DMA,
        ],
        compiler_params=pltpu.CompilerParams(use_tc_tiling_on_sc=True),
    )
    return fn(values.astype(jnp.float32), inputs.T).T
